# Initial kernel scaffold; baseline (speedup 1.0000x reference)
import jax
import jax.numpy as jnp
from jax.experimental import pallas as pl


def kernel(vertices, alpha, d0, w1, b1, dir1, w2, b2, dir2, w3, b3, dir3, w4, b4, dir4, w5, b5, dir5, cW1, cb1, cg, cbe, cW2, cb2, dW1, db1, dg, dbe, dW2, db2):
    raise NotImplementedError("write your pallas kernel here")


# speedup vs baseline: 2.3920x; 2.3920x over previous
import math

import jax
import jax.numpy as jnp
import numpy as np
from jax import lax
from jax.experimental import pallas as pl
from jax.experimental.pallas import tpu as pltpu

_K = 20
_HI = lax.Precision.HIGHEST
_F32 = jnp.float32


def _mm(a, b):
    return lax.dot_general(a, b, (((1,), (0,)), ((), ())), precision=_HI,
                           preferred_element_type=_F32)


def _norm_dirs(d):
    n = jnp.sqrt(jnp.sum(d * d, axis=0, keepdims=True))
    return d / jnp.maximum(n, 1e-12)


def _norm_rows(x):
    n = jnp.sqrt(jnp.sum(x * x, axis=1, keepdims=True))
    return x / jnp.maximum(n, 1e-12)


def _theta(nd, dn):
    t = nd[:, 0:1] * dn[0:1, :] + nd[:, 1:2] * dn[1:2, :] + nd[:, 2:3] * dn[2:3, :]
    return jnp.maximum(t, 0.0)


def _pdist(vb, vt):
    quad_col = jnp.sum(vt * vt, axis=0, keepdims=True)
    quad_row = jnp.sum(vb * vb, axis=1, keepdims=True)
    inner = (vb[:, 0:1] * vt[0:1, :] + vb[:, 1:2] * vt[1:2, :]
             + vb[:, 2:3] * vt[2:3, :])
    return -2.0 * inner + quad_col + quad_row


def _col(mat, j, width):
    it = lax.broadcasted_iota(jnp.int32, (mat.shape[0], width), 1)
    return jnp.sum(jnp.where(it == j, mat, 0), axis=1, keepdims=True)


def _conv_neighbor_max(ni, va, vb, sup_table, dirn, out_c):
    rows = vb.shape[0]
    vsrc = va.shape[0]
    cit = lax.broadcasted_iota(jnp.int32, (rows, vsrc), 1)

    def body(j, acc):
        nij = _col(ni, j, _K)
        oh = (cit == nij).astype(_F32)
        nbr = _mm(oh, va)
        ndn = _norm_rows(nbr - vb)
        th = _theta(ndn, dirn)
        sup = _mm(oh, sup_table)
        return jnp.maximum(acc, th * sup)

    init = jnp.full((rows, out_c), -jnp.inf, _F32)
    return lax.fori_loop(0, _K, body, init)


def _pool_max(pni, table):
    rows = pni.shape[0]
    cit = lax.broadcasted_iota(jnp.int32, (rows, table.shape[0]), 1)

    def body(j, acc):
        nij = _col(pni, j, 4)
        oh = (cit == nij).astype(_F32)
        return jnp.maximum(acc, _mm(oh, table))

    init = jnp.full((rows, table.shape[1]), -jnp.inf, _F32)
    return lax.fori_loop(0, 4, body, init)


def _topk_ni(scr, dist, k):
    rows, cols = dist.shape
    cit = lax.broadcasted_iota(jnp.int32, (rows, cols), 1)
    lane = lax.broadcasted_iota(jnp.int32, (rows, 32), 1)
    scr[...] = dist

    def body(i, ni_acc):
        d = scr[...]
        m = jnp.min(d, axis=1, keepdims=True)
        idx = jnp.min(jnp.where(d == m, cit, cols), axis=1, keepdims=True)
        scr[...] = jnp.where(cit == idx, jnp.inf, d)
        return jnp.where(lane == i - 1, idx, ni_acc)

    ni_acc = lax.fori_loop(0, k + 1, body, jnp.zeros((rows, 32), jnp.int32))
    return ni_acc[:, :k]


def _k1_body(vb_ref, va_ref, vt_ref, d0_ref, ni_ref, fm0_ref, scr):
    vb = vb_ref[0]
    va = va_ref[0]
    vt = vt_ref[0]
    rows, cols = vb.shape[0], va.shape[0]
    dist = _pdist(vb, vt)
    cit = lax.broadcasted_iota(jnp.int32, (rows, cols), 1)
    lane = lax.broadcasted_iota(jnp.int32, (rows, 32), 1)
    d0n = _norm_dirs(d0_ref[...])
    scr[...] = dist

    def body(i, carry):
        ni_acc, acc0 = carry
        d = scr[...]
        m = jnp.min(d, axis=1, keepdims=True)
        idx = jnp.min(jnp.where(d == m, cit, cols), axis=1, keepdims=True)
        sel = cit == idx
        scr[...] = jnp.where(sel, jnp.inf, d)
        oh = sel.astype(_F32)
        nbr = _mm(oh, va)
        ndn = _norm_rows(nbr - vb)
        acc0 = jnp.maximum(acc0, _theta(ndn, d0n))
        ni_acc = jnp.where(lane == i - 1, idx, ni_acc)
        return ni_acc, acc0

    ni_acc, acc0 = lax.fori_loop(
        0, _K + 1, body,
        (jnp.zeros((rows, 32), jnp.int32), jnp.zeros((rows, 16), _F32)))
    ni_ref[0] = ni_acc[:, :_K]
    fm0_ref[0] = jnp.maximum(acc0, 0.0)


def _k2_body(va_ref, vb_ref, fm0a_ref, fm0b_ref, ni_ref, w_ref, b_ref,
             dir_ref, fm1_ref):
    va = va_ref[0]
    vb = vb_ref[0]
    fa = fm0a_ref[0]
    fb = fm0b_ref[0]
    nib = ni_ref[0]
    w = w_ref[...]
    b = b_ref[...]
    sup_table = _mm(fa, w[:, 32:]) + b[:, 32:]
    center = _mm(fb, w[:, :32]) + b[:, :32]
    dn = _norm_dirs(dir_ref[...])
    acc = _conv_neighbor_max(nib, va, vb, sup_table, dn, 32)
    fm1_ref[0] = jnp.maximum(center + acc, 0.0)


def _k4_body(vb_ref, vt_ref, fm1_ref, pni_ref, w2_ref, b2_ref, dir2_ref,
             w3_ref, b3_ref, dir3_ref, fm3_ref, ni2_ref, scr):
    vb = vb_ref[0]
    vt = vt_ref[0]
    fm1 = fm1_ref[0]
    pni = pni_ref[0]

    fmp = _pool_max(pni, fm1)
    ni = _topk_ni(scr, _pdist(vb, vt), _K)

    t2 = _mm(fmp, w2_ref[...]) + b2_ref[...]
    d2n = _norm_dirs(dir2_ref[...])
    acc = _conv_neighbor_max(ni, vb, vb, t2[:, 64:], d2n, 64)
    fm2 = jnp.maximum(t2[:, :64] + acc, 0.0)

    mp2 = jnp.max(fm2, axis=0, keepdims=True)
    fm2c = jnp.concatenate([fm2, jnp.broadcast_to(mp2, fm2.shape)], axis=1)

    t3 = _mm(fm2c, w3_ref[...]) + b3_ref[...]
    d3n = _norm_dirs(dir3_ref[...])
    acc = _conv_neighbor_max(ni, vb, vb, t3[:, 256:], d3n, 256)
    fm3_ref[0] = jnp.maximum(t3[:, :256] + acc, 0.0)
    ni2_ref[0] = ni


def _k5_body(vb_ref, vt_ref, fm3_ref, pni_ref, w4_ref, b4_ref, dir4_ref,
             w5_ref, b5_ref, dir5_ref, fg_ref, scr):
    vb = vb_ref[0]
    vt = vt_ref[0]
    fm3 = fm3_ref[0]
    pni = pni_ref[0]

    fmp = _pool_max(pni, fm3)
    ni = _topk_ni(scr, _pdist(vb, vt), _K)

    t4 = _mm(fmp, w4_ref[...]) + b4_ref[...]
    d4n = _norm_dirs(dir4_ref[...])
    acc = _conv_neighbor_max(ni, vb, vb, t4[:, 256:], d4n, 256)
    fm4 = jnp.maximum(t4[:, :256] + acc, 0.0)

    mp4 = jnp.max(fm4, axis=0, keepdims=True)
    fm4c = jnp.concatenate([fm4, jnp.broadcast_to(mp4, fm4.shape)], axis=1)

    t5 = _mm(fm4c, w5_ref[...]) + b5_ref[...]
    d5n = _norm_dirs(dir5_ref[...])
    acc = _conv_neighbor_max(ni, vb, vb, t5[:, 1024:], d5n, 1024)
    fm5 = jnp.maximum(t5[:, :1024] + acc, 0.0)
    fg_ref[0] = jnp.max(fm5, axis=0, keepdims=True)


def _k6_body(fg_ref, cW1_ref, cb1_ref, cg_ref, cbe_ref, cW2_ref, cb2_ref,
             dW1_ref, db1_ref, dg_ref, dbe_ref, dW2_ref, db2_ref,
             co_ref, do_ref):
    fg = fg_ref[...]
    inv = 1.0 / math.sqrt(1.0 + 1e-5)

    def _head(W1, b1, g, be, W2, b2):
        h = _mm(fg, W1) + b1
        h = g * h * inv + be
        h = jnp.maximum(h, 0.0)
        lg = _mm(h, W2) + b2
        m = jnp.max(lg, axis=1, keepdims=True)
        s = lg - m
        return s - jnp.log(jnp.sum(jnp.exp(s), axis=1, keepdims=True))

    co_ref[...] = _head(cW1_ref[...], cb1_ref[...], cg_ref[...],
                        cbe_ref[...], cW2_ref[...], cb2_ref[...])
    do_ref[...] = _head(dW1_ref[...], db1_ref[...], dg_ref[...],
                        dbe_ref[...], dW2_ref[...], db2_ref[...])


_PERM1 = np.asarray(jax.random.permutation(jax.random.key(1), 2048))[:512]
_PERM2 = np.asarray(jax.random.permutation(jax.random.key(2), 512))[:128]


def _full(shape):
    return pl.BlockSpec(shape, lambda b, *_: (0,) * len(shape))


def kernel(vertices, alpha, d0, w1, b1, dir1, w2, b2, dir2, w3, b3, dir3,
           w4, b4, dir4, w5, b5, dir5, cW1, cb1, cg, cbe, cW2, cb2,
           dW1, db1, dg, dbe, dW2, db2):
    del alpha
    bs, V, _ = vertices.shape
    Rb = 256
    nb = V // Rb
    f32 = jnp.float32
    vT = jnp.swapaxes(vertices, 1, 2)
    b1r, b2r, b3r, b4r, b5r = (x.reshape(1, -1) for x in (b1, b2, b3, b4, b5))

    ni1, fm0 = pl.pallas_call(
        _k1_body,
        grid=(bs, nb),
        in_specs=[
            pl.BlockSpec((1, Rb, 3), lambda b, r: (b, r, 0)),
            pl.BlockSpec((1, V, 3), lambda b, r: (b, 0, 0)),
            pl.BlockSpec((1, 3, V), lambda b, r: (b, 0, 0)),
            pl.BlockSpec((3, 16), lambda b, r: (0, 0)),
        ],
        out_specs=[
            pl.BlockSpec((1, Rb, _K), lambda b, r: (b, r, 0)),
            pl.BlockSpec((1, Rb, 16), lambda b, r: (b, r, 0)),
        ],
        out_shape=[
            jax.ShapeDtypeStruct((bs, V, _K), jnp.int32),
            jax.ShapeDtypeStruct((bs, V, 16), f32),
        ],
        scratch_shapes=[pltpu.VMEM((Rb, V), f32)],
    )(vertices, vertices, vT, d0)

    fm1 = pl.pallas_call(
        _k2_body,
        grid=(bs, nb),
        in_specs=[
            pl.BlockSpec((1, V, 3), lambda b, r: (b, 0, 0)),
            pl.BlockSpec((1, Rb, 3), lambda b, r: (b, r, 0)),
            pl.BlockSpec((1, V, 16), lambda b, r: (b, 0, 0)),
            pl.BlockSpec((1, Rb, 16), lambda b, r: (b, r, 0)),
            pl.BlockSpec((1, Rb, _K), lambda b, r: (b, r, 0)),
            pl.BlockSpec((16, 64), lambda b, r: (0, 0)),
            pl.BlockSpec((1, 64), lambda b, r: (0, 0)),
            pl.BlockSpec((3, 32), lambda b, r: (0, 0)),
        ],
        out_specs=pl.BlockSpec((1, Rb, 32), lambda b, r: (b, r, 0)),
        out_shape=jax.ShapeDtypeStruct((bs, V, 32), f32),
    )(vertices, vertices, fm0, fm0, ni1, w1, b1r, dir1)

    V2 = 512
    vert2 = vertices[:, _PERM1, :]
    v2T = jnp.swapaxes(vert2, 1, 2)
    pni1 = ni1[:, _PERM1, :4]

    fm3, ni2 = pl.pallas_call(
        _k4_body,
        grid=(bs,),
        in_specs=[
            pl.BlockSpec((1, V2, 3), lambda b: (b, 0, 0)),
            pl.BlockSpec((1, 3, V2), lambda b: (b, 0, 0)),
            pl.BlockSpec((1, V, 32), lambda b: (b, 0, 0)),
            pl.BlockSpec((1, V2, 4), lambda b: (b, 0, 0)),
            _full((32, 128)), _full((1, 128)), _full((3, 64)),
            _full((128, 512)), _full((1, 512)), _full((3, 256)),
        ],
        out_specs=[
            pl.BlockSpec((1, V2, 256), lambda b: (b, 0, 0)),
            pl.BlockSpec((1, V2, _K), lambda b: (b, 0, 0)),
        ],
        out_shape=[
            jax.ShapeDtypeStruct((bs, V2, 256), f32),
            jax.ShapeDtypeStruct((bs, V2, _K), jnp.int32),
        ],
        scratch_shapes=[pltpu.VMEM((V2, V2), f32)],
    )(vert2, v2T, fm1, pni1, w2, b2r, dir2, w3, b3r, dir3)

    V3 = 128
    vert3 = vert2[:, _PERM2, :]
    v3T = jnp.swapaxes(vert3, 1, 2)
    pni2 = ni2[:, _PERM2, :4]

    fg = pl.pallas_call(
        _k5_body,
        grid=(bs,),
        in_specs=[
            pl.BlockSpec((1, V3, 3), lambda b: (b, 0, 0)),
            pl.BlockSpec((1, 3, V3), lambda b: (b, 0, 0)),
            pl.BlockSpec((1, V2, 256), lambda b: (b, 0, 0)),
            pl.BlockSpec((1, V3, 4), lambda b: (b, 0, 0)),
            _full((256, 512)), _full((1, 512)), _full((3, 256)),
            _full((512, 2048)), _full((1, 2048)), _full((3, 1024)),
        ],
        out_specs=pl.BlockSpec((1, 1, 1024), lambda b: (b, 0, 0)),
        out_shape=jax.ShapeDtypeStruct((bs, 1, 1024), f32),
        scratch_shapes=[pltpu.VMEM((V3, V3), f32)],
    )(vert3, v3T, fm3, pni2, w4, b4r, dir4, w5, b5r, dir5)

    fgr = fg.reshape(bs, 1024)
    co, do = pl.pallas_call(
        _k6_body,
        in_specs=[
            pl.BlockSpec((bs, 1024), lambda: (0, 0)),
            pl.BlockSpec((1024, 256), lambda: (0, 0)),
            pl.BlockSpec((1, 256), lambda: (0, 0)),
            pl.BlockSpec((1, 256), lambda: (0, 0)),
            pl.BlockSpec((1, 256), lambda: (0, 0)),
            pl.BlockSpec((256, 3), lambda: (0, 0)),
            pl.BlockSpec((1, 3), lambda: (0, 0)),
            pl.BlockSpec((1024, 256), lambda: (0, 0)),
            pl.BlockSpec((1, 256), lambda: (0, 0)),
            pl.BlockSpec((1, 256), lambda: (0, 0)),
            pl.BlockSpec((1, 256), lambda: (0, 0)),
            pl.BlockSpec((256, 2), lambda: (0, 0)),
            pl.BlockSpec((1, 2), lambda: (0, 0)),
        ],
        out_specs=[
            pl.BlockSpec((bs, 3), lambda: (0, 0)),
            pl.BlockSpec((bs, 2), lambda: (0, 0)),
        ],
        out_shape=[
            jax.ShapeDtypeStruct((bs, 3), f32),
            jax.ShapeDtypeStruct((bs, 2), f32),
        ],
    )(fgr, cW1, cb1.reshape(1, -1), cg.reshape(1, -1), cbe.reshape(1, -1),
      cW2, cb2.reshape(1, -1), dW1, db1.reshape(1, -1), dg.reshape(1, -1),
      dbe.reshape(1, -1), dW2, db2.reshape(1, -1))
    return co, do


# speedup vs baseline: 11.8365x; 4.9483x over previous
import math

import jax
import jax.numpy as jnp
import numpy as np
from jax import lax
from jax.experimental import pallas as pl
from jax.experimental.pallas import tpu as pltpu

_K = 20
_F32 = jnp.float32


def _mm(a, b):
    return lax.dot_general(a, b, (((1,), (0,)), ((), ())),
                           preferred_element_type=_F32)


def _norm_dirs(d):
    n = jnp.sqrt(jnp.sum(d * d, axis=0, keepdims=True))
    return d / jnp.maximum(n, 1e-12)


def _norm_rows(x):
    n = jnp.sqrt(jnp.sum(x * x, axis=1, keepdims=True))
    return x / jnp.maximum(n, 1e-12)


def _theta(nd, dn):
    t = nd[:, 0:1] * dn[0:1, :] + nd[:, 1:2] * dn[1:2, :] + nd[:, 2:3] * dn[2:3, :]
    return jnp.maximum(t, 0.0)


def _pdist(vb, vt):
    quad_col = jnp.sum(vt * vt, axis=0, keepdims=True)
    quad_row = jnp.sum(vb * vb, axis=1, keepdims=True)
    inner = (vb[:, 0:1] * vt[0:1, :] + vb[:, 1:2] * vt[1:2, :]
             + vb[:, 2:3] * vt[2:3, :])
    return -2.0 * inner + quad_col + quad_row


def _conv_neighbor_max(ni, table, vb, dirn, out_c):
    rows = vb.shape[0]
    vsrc = table.shape[0]
    cit = lax.broadcasted_iota(jnp.int32, (rows, vsrc), 1)
    acc = None
    for j in range(_K):
        nij = ni[:, j:j + 1]
        oh = (cit == nij).astype(_F32)
        g = _mm(oh, table)
        ndn = _norm_rows(g[:, out_c:out_c + 3] - vb)
        v = _theta(ndn, dirn) * g[:, :out_c]
        acc = v if acc is None else jnp.maximum(acc, v)
    return acc


def _pool_max(pni, table):
    rows = pni.shape[0]
    cit = lax.broadcasted_iota(jnp.int32, (rows, table.shape[0]), 1)
    acc = None
    for j in range(4):
        oh = (cit == pni[:, j:j + 1]).astype(_F32)
        g = _mm(oh, table)
        acc = g if acc is None else jnp.maximum(acc, g)
    return acc


def _topk_ni(dist, k):
    rows, cols = dist.shape
    cit = lax.broadcasted_iota(jnp.int32, (rows, cols), 1)
    lane = lax.broadcasted_iota(jnp.int32, (rows, 32), 1)
    d = dist
    ni_acc = jnp.zeros((rows, 32), jnp.int32)
    for i in range(k + 1):
        idx = jnp.argmin(d, axis=1).astype(jnp.int32)[:, None]
        d = jnp.where(cit == idx, jnp.inf, d)
        if i > 0:
            ni_acc = jnp.where(lane == i - 1, idx, ni_acc)
    return ni_acc[:, :k]


def _k1_body(vb_ref, va_ref, vt_ref, d0_ref, ni_ref, fm0_ref):
    vb = vb_ref[0]
    va = va_ref[0]
    vt = vt_ref[0]
    rows, cols = vb.shape[0], va.shape[0]
    d = _pdist(vb, vt)
    cit = lax.broadcasted_iota(jnp.int32, (rows, cols), 1)
    lane = lax.broadcasted_iota(jnp.int32, (rows, 32), 1)
    d0n = _norm_dirs(d0_ref[...])

    ni_acc = jnp.zeros((rows, 32), jnp.int32)
    acc0 = jnp.zeros((rows, 16), _F32)
    for i in range(_K + 1):
        m = jnp.min(d, axis=1, keepdims=True)
        idx = jnp.min(jnp.where(d == m, cit, cols), axis=1, keepdims=True)
        sel = cit == idx
        d = jnp.where(sel, jnp.inf, d)
        oh = sel.astype(_F32)
        nbr = _mm(oh, va)
        ndn = _norm_rows(nbr - vb)
        acc0 = jnp.maximum(acc0, _theta(ndn, d0n))
        if i > 0:
            ni_acc = jnp.where(lane == i - 1, idx, ni_acc)
    ni_ref[0] = ni_acc[:, :_K]
    fm0_ref[0] = acc0


def _k2_body(va_ref, vb_ref, fm0a_ref, fm0b_ref, ni_ref, w_ref, b_ref,
             dir_ref, fm1_ref):
    va = va_ref[0]
    vb = vb_ref[0]
    fa = fm0a_ref[0]
    fb = fm0b_ref[0]
    nib = ni_ref[0]
    w = w_ref[...]
    b = b_ref[...]
    sup_table = _mm(fa, w[:, 32:]) + b[:, 32:]
    center = _mm(fb, w[:, :32]) + b[:, :32]
    dn = _norm_dirs(dir_ref[...])
    table = jnp.concatenate([sup_table, va], axis=1)
    acc = _conv_neighbor_max(nib, table, vb, dn, 32)
    fm1_ref[0] = jnp.maximum(center + acc, 0.0)


def _k4_body(vb_ref, vt_ref, fm1_ref, pni_ref, w2_ref, b2_ref, dir2_ref,
             w3_ref, b3_ref, dir3_ref, fm3_ref, ni2_ref):
    vb = vb_ref[0]
    vt = vt_ref[0]
    fm1 = fm1_ref[0]
    pni = pni_ref[0]

    fmp = _pool_max(pni, fm1)
    ni = _topk_ni(_pdist(vb, vt), _K)

    t2 = _mm(fmp, w2_ref[...]) + b2_ref[...]
    d2n = _norm_dirs(dir2_ref[...])
    tbl2 = jnp.concatenate([t2[:, 64:], vb], axis=1)
    acc = _conv_neighbor_max(ni, tbl2, vb, d2n, 64)
    fm2 = jnp.maximum(t2[:, :64] + acc, 0.0)

    mp2 = jnp.max(fm2, axis=0, keepdims=True)
    fm2c = jnp.concatenate([fm2, jnp.broadcast_to(mp2, fm2.shape)], axis=1)

    t3 = _mm(fm2c, w3_ref[...]) + b3_ref[...]
    d3n = _norm_dirs(dir3_ref[...])
    tbl3 = jnp.concatenate([t3[:, 256:], vb], axis=1)
    acc = _conv_neighbor_max(ni, tbl3, vb, d3n, 256)
    fm3_ref[0] = jnp.maximum(t3[:, :256] + acc, 0.0)
    ni2_ref[0] = ni


def _k5_body(vb_ref, vt_ref, fm3_ref, pni_ref, w4_ref, b4_ref, dir4_ref,
             w5_ref, b5_ref, dir5_ref, fg_ref):
    vb = vb_ref[0]
    vt = vt_ref[0]
    fm3 = fm3_ref[0]
    pni = pni_ref[0]

    fmp = _pool_max(pni, fm3)
    ni = _topk_ni(_pdist(vb, vt), _K)

    t4 = _mm(fmp, w4_ref[...]) + b4_ref[...]
    d4n = _norm_dirs(dir4_ref[...])
    tbl4 = jnp.concatenate([t4[:, 256:], vb], axis=1)
    acc = _conv_neighbor_max(ni, tbl4, vb, d4n, 256)
    fm4 = jnp.maximum(t4[:, :256] + acc, 0.0)

    mp4 = jnp.max(fm4, axis=0, keepdims=True)
    fm4c = jnp.concatenate([fm4, jnp.broadcast_to(mp4, fm4.shape)], axis=1)

    t5 = _mm(fm4c, w5_ref[...]) + b5_ref[...]
    d5n = _norm_dirs(dir5_ref[...])
    tbl5 = jnp.concatenate([t5[:, 1024:], vb], axis=1)
    acc = _conv_neighbor_max(ni, tbl5, vb, d5n, 1024)
    fm5 = jnp.maximum(t5[:, :1024] + acc, 0.0)
    fg_ref[0] = jnp.max(fm5, axis=0, keepdims=True)


def _k6_body(fg_ref, cW1_ref, cb1_ref, cg_ref, cbe_ref, cW2_ref, cb2_ref,
             dW1_ref, db1_ref, dg_ref, dbe_ref, dW2_ref, db2_ref,
             co_ref, do_ref):
    fg = fg_ref[...]
    inv = 1.0 / math.sqrt(1.0 + 1e-5)

    def _head(W1, b1, g, be, W2, b2):
        h = _mm(fg, W1) + b1
        h = g * h * inv + be
        h = jnp.maximum(h, 0.0)
        lg = _mm(h, W2) + b2
        m = jnp.max(lg, axis=1, keepdims=True)
        s = lg - m
        return s - jnp.log(jnp.sum(jnp.exp(s), axis=1, keepdims=True))

    co_ref[...] = _head(cW1_ref[...], cb1_ref[...], cg_ref[...],
                        cbe_ref[...], cW2_ref[...], cb2_ref[...])
    do_ref[...] = _head(dW1_ref[...], db1_ref[...], dg_ref[...],
                        dbe_ref[...], dW2_ref[...], db2_ref[...])


_PERM1 = np.array([
    1308,98,1494,1367,1392,726,410,1311,1631,1841,360,1261,1990,139,467,
    1964,1122,1547,739,892,198,610,1721,1669,1822,1265,1502,1965,858,292,
    210,965,1029,1185,1888,1968,688,1230,941,158,352,539,294,795,26,919,
    120,853,216,340,1356,1324,1164,236,13,482,414,1168,1726,1854,873,883,
    1909,1982,73,90,107,953,114,752,1388,1274,1556,702,88,226,868,1707,49,
    488,1761,1248,423,442,641,1767,1755,1012,1570,1598,0,1111,855,1142,
    1713,601,529,34,1522,1187,305,1087,202,948,751,443,806,206,1067,803,
    637,250,1224,51,1147,1772,533,457,661,1402,863,242,1534,1366,666,1756,
    1445,622,709,437,519,142,1847,1658,95,1700,1863,1381,1042,991,75,357,
    794,1549,495,1614,1451,525,1262,1030,1925,1904,404,1680,1942,200,385,
    1134,239,2003,39,619,1327,459,680,1475,432,694,1518,141,588,685,1660,
    122,715,1783,35,1139,274,797,1346,608,670,2001,362,409,1428,978,658,
    1543,1341,1343,1708,958,1843,1440,1406,378,1719,341,123,1306,116,1107,
    1967,21,1781,1896,1056,1026,551,1450,1926,1711,370,649,268,307,2034,
    2011,168,1500,1739,2000,1218,503,1325,748,1616,1193,1605,1437,1319,
    1595,1427,252,1481,1851,1116,1102,902,4,1053,273,1098,600,1453,386,
    1927,1734,1859,1974,1221,1683,763,1532,1724,365,829,732,1277,1831,
    1439,586,890,1836,96,1656,581,230,900,1943,1498,416,1,1794,1106,152,
    520,827,969,1206,245,1624,1741,452,1803,129,549,76,924,857,1931,884,
    623,1174,558,862,1826,315,448,361,754,1559,568,1586,254,1035,952,81,
    769,41,1144,2018,501,248,1268,382,575,1899,1104,2019,1213,1489,338,
    1045,973,280,1121,255,1099,1579,954,1555,1061,921,89,1090,1569,422,
    1635,400,93,1241,1373,407,1079,205,209,363,1988,839,636,871,647,1796,
    698,1048,615,218,1186,894,434,1393,767,1088,672,1084,47,692,293,66,
    1845,70,756,174,222,1457,2014,532,1520,1821,1645,1077,1488,1149,793,
    1097,1001,1671,1618,1505,1811,1156,387,1685,1674,426,1008,128,617,882,
    980,648,1524,1996,1938,1597,194,1834,1467,1949,1289,1743,312,1833,
    1615,1305,1027,1095,1177,598,1212,393,1897,1986,1485,917,285,1940,321,
    347,1566,950,1966,1062,611,728,1257,11,1426,1307,1676,435,1873,984,
    1696,1083,1215,741,1960,625,1419,845,1345,1535,308,309,1171,572,779,
    785,1571,824,557,1916,1359,578,156,771,440,1058,430,706,1805,9,1123,
    1023,1145,244,1663,1161,1878,1934,1880,1483,1544,997,1234,681,1094,
    727,1179,1376,1953,492,1499,995,718,736,333,1792,1390,1000,1868,1253,
    1205,957,1014,345,787,961,
], dtype=np.int32)
_PERM2 = np.array([
    135,164,319,83,387,107,91,503,52,58,2,379,450,238,156,501,59,467,73,
    15,388,177,449,375,394,498,284,225,53,129,243,136,415,196,63,10,484,
    239,359,455,185,444,244,497,158,181,198,422,474,138,113,393,67,29,389,
    94,396,162,456,62,163,499,260,468,464,159,229,311,179,271,248,174,191,
    273,426,241,92,224,365,117,295,383,391,126,446,505,508,251,110,459,98,
    309,81,451,441,373,352,250,66,476,349,438,285,431,482,55,478,343,249,
    294,9,85,28,469,194,124,259,448,80,386,18,480,235,176,45,31,408,418,
], dtype=np.int32)


def _full(shape):
    return pl.BlockSpec(shape, lambda b, *_: (0,) * len(shape))


def kernel(vertices, alpha, d0, w1, b1, dir1, w2, b2, dir2, w3, b3, dir3,
           w4, b4, dir4, w5, b5, dir5, cW1, cb1, cg, cbe, cW2, cb2,
           dW1, db1, dg, dbe, dW2, db2):
    del alpha
    bs, V, _ = vertices.shape
    Rb = 256
    nb = V // Rb
    f32 = jnp.float32
    vT = jnp.swapaxes(vertices, 1, 2)
    b1r, b2r, b3r, b4r, b5r = (x.reshape(1, -1) for x in (b1, b2, b3, b4, b5))

    ni1, fm0 = pl.pallas_call(
        _k1_body,
        grid=(bs, nb),
        in_specs=[
            pl.BlockSpec((1, Rb, 3), lambda b, r: (b, r, 0)),
            pl.BlockSpec((1, V, 3), lambda b, r: (b, 0, 0)),
            pl.BlockSpec((1, 3, V), lambda b, r: (b, 0, 0)),
            pl.BlockSpec((3, 16), lambda b, r: (0, 0)),
        ],
        out_specs=[
            pl.BlockSpec((1, Rb, _K), lambda b, r: (b, r, 0)),
            pl.BlockSpec((1, Rb, 16), lambda b, r: (b, r, 0)),
        ],
        out_shape=[
            jax.ShapeDtypeStruct((bs, V, _K), jnp.int32),
            jax.ShapeDtypeStruct((bs, V, 16), f32),
        ],
    )(vertices, vertices, vT, d0)

    fm1 = pl.pallas_call(
        _k2_body,
        grid=(bs, nb),
        in_specs=[
            pl.BlockSpec((1, V, 3), lambda b, r: (b, 0, 0)),
            pl.BlockSpec((1, Rb, 3), lambda b, r: (b, r, 0)),
            pl.BlockSpec((1, V, 16), lambda b, r: (b, 0, 0)),
            pl.BlockSpec((1, Rb, 16), lambda b, r: (b, r, 0)),
            pl.BlockSpec((1, Rb, _K), lambda b, r: (b, r, 0)),
            pl.BlockSpec((16, 64), lambda b, r: (0, 0)),
            pl.BlockSpec((1, 64), lambda b, r: (0, 0)),
            pl.BlockSpec((3, 32), lambda b, r: (0, 0)),
        ],
        out_specs=pl.BlockSpec((1, Rb, 32), lambda b, r: (b, r, 0)),
        out_shape=jax.ShapeDtypeStruct((bs, V, 32), f32),
    )(vertices, vertices, fm0, fm0, ni1, w1, b1r, dir1)

    V2 = 512
    vert2 = vertices[:, _PERM1, :]
    v2T = jnp.swapaxes(vert2, 1, 2)
    pni1 = ni1[:, _PERM1, :4]

    fm3, ni2 = pl.pallas_call(
        _k4_body,
        grid=(bs,),
        in_specs=[
            pl.BlockSpec((1, V2, 3), lambda b: (b, 0, 0)),
            pl.BlockSpec((1, 3, V2), lambda b: (b, 0, 0)),
            pl.BlockSpec((1, V, 32), lambda b: (b, 0, 0)),
            pl.BlockSpec((1, V2, 4), lambda b: (b, 0, 0)),
            _full((32, 128)), _full((1, 128)), _full((3, 64)),
            _full((128, 512)), _full((1, 512)), _full((3, 256)),
        ],
        out_specs=[
            pl.BlockSpec((1, V2, 256), lambda b: (b, 0, 0)),
            pl.BlockSpec((1, V2, _K), lambda b: (b, 0, 0)),
        ],
        out_shape=[
            jax.ShapeDtypeStruct((bs, V2, 256), f32),
            jax.ShapeDtypeStruct((bs, V2, _K), jnp.int32),
        ],
    )(vert2, v2T, fm1, pni1, w2, b2r, dir2, w3, b3r, dir3)

    V3 = 128
    vert3 = vert2[:, _PERM2, :]
    v3T = jnp.swapaxes(vert3, 1, 2)
    pni2 = ni2[:, _PERM2, :4]

    fg = pl.pallas_call(
        _k5_body,
        grid=(bs,),
        in_specs=[
            pl.BlockSpec((1, V3, 3), lambda b: (b, 0, 0)),
            pl.BlockSpec((1, 3, V3), lambda b: (b, 0, 0)),
            pl.BlockSpec((1, V2, 256), lambda b: (b, 0, 0)),
            pl.BlockSpec((1, V3, 4), lambda b: (b, 0, 0)),
            _full((256, 512)), _full((1, 512)), _full((3, 256)),
            _full((512, 2048)), _full((1, 2048)), _full((3, 1024)),
        ],
        out_specs=pl.BlockSpec((1, 1, 1024), lambda b: (b, 0, 0)),
        out_shape=jax.ShapeDtypeStruct((bs, 1, 1024), f32),
    )(vert3, v3T, fm3, pni2, w4, b4r, dir4, w5, b5r, dir5)

    fgr = fg.reshape(bs, 1024)
    co, do = pl.pallas_call(
        _k6_body,
        in_specs=[
            pl.BlockSpec((bs, 1024), lambda: (0, 0)),
            pl.BlockSpec((1024, 256), lambda: (0, 0)),
            pl.BlockSpec((1, 256), lambda: (0, 0)),
            pl.BlockSpec((1, 256), lambda: (0, 0)),
            pl.BlockSpec((1, 256), lambda: (0, 0)),
            pl.BlockSpec((256, 3), lambda: (0, 0)),
            pl.BlockSpec((1, 3), lambda: (0, 0)),
            pl.BlockSpec((1024, 256), lambda: (0, 0)),
            pl.BlockSpec((1, 256), lambda: (0, 0)),
            pl.BlockSpec((1, 256), lambda: (0, 0)),
            pl.BlockSpec((1, 256), lambda: (0, 0)),
            pl.BlockSpec((256, 2), lambda: (0, 0)),
            pl.BlockSpec((1, 2), lambda: (0, 0)),
        ],
        out_specs=[
            pl.BlockSpec((bs, 3), lambda: (0, 0)),
            pl.BlockSpec((bs, 2), lambda: (0, 0)),
        ],
        out_shape=[
            jax.ShapeDtypeStruct((bs, 3), f32),
            jax.ShapeDtypeStruct((bs, 2), f32),
        ],
    )(fgr, cW1, cb1.reshape(1, -1), cg.reshape(1, -1), cbe.reshape(1, -1),
      cW2, cb2.reshape(1, -1), dW1, db1.reshape(1, -1), dg.reshape(1, -1),
      dbe.reshape(1, -1), dW2, db2.reshape(1, -1))
    return co, do


# speedup vs baseline: 12.4194x; 1.0492x over previous
import math

import jax
import jax.numpy as jnp
import numpy as np
from jax import lax
from jax.experimental import pallas as pl
from jax.experimental.pallas import tpu as pltpu

_K = 20
_F32 = jnp.float32


def _mm(a, b):
    return lax.dot_general(a, b, (((1,), (0,)), ((), ())),
                           preferred_element_type=_F32)


def _norm_dirs(d):
    n = jnp.sqrt(jnp.sum(d * d, axis=0, keepdims=True))
    return d / jnp.maximum(n, 1e-12)


def _norm_rows(x):
    n = jnp.sqrt(jnp.sum(x * x, axis=1, keepdims=True))
    return x / jnp.maximum(n, 1e-12)


def _theta(nd, dn):
    t = nd[:, 0:1] * dn[0:1, :] + nd[:, 1:2] * dn[1:2, :] + nd[:, 2:3] * dn[2:3, :]
    return jnp.maximum(t, 0.0)


def _pdist(vb, vt):
    quad_col = jnp.sum(vt * vt, axis=0, keepdims=True)
    quad_row = jnp.sum(vb * vb, axis=1, keepdims=True)
    inner = (vb[:, 0:1] * vt[0:1, :] + vb[:, 1:2] * vt[1:2, :]
             + vb[:, 2:3] * vt[2:3, :])
    return -2.0 * inner + quad_col + quad_row


def _conv_neighbor_max(ni, table, vb, dirn, out_c, ndns=None):
    rows = vb.shape[0]
    vsrc = table.shape[0]
    cit = lax.broadcasted_iota(jnp.int32, (rows, vsrc), 1)
    acc = None
    compute_ndn = ndns is None
    if compute_ndn:
        ndns = []
    for j in range(_K):
        nij = ni[:, j:j + 1]
        oh = (cit == nij).astype(_F32)
        g = _mm(oh, table)
        if compute_ndn:
            ndn = _norm_rows(g[:, out_c:out_c + 3] - vb)
            ndns.append(ndn)
        else:
            ndn = ndns[j]
        v = _theta(ndn, dirn) * g[:, :out_c]
        acc = v if acc is None else jnp.maximum(acc, v)
    return acc, ndns


def _pool_max(pni, table):
    rows = pni.shape[0]
    cit = lax.broadcasted_iota(jnp.int32, (rows, table.shape[0]), 1)
    acc = None
    for j in range(4):
        oh = (cit == pni[:, j:j + 1]).astype(_F32)
        g = _mm(oh, table)
        acc = g if acc is None else jnp.maximum(acc, g)
    return acc


def _topk_ni(dist, k):
    rows, cols = dist.shape
    cit = lax.broadcasted_iota(jnp.int32, (rows, cols), 1)
    lane = lax.broadcasted_iota(jnp.int32, (rows, 32), 1)
    d = dist
    ni_acc = jnp.zeros((rows, 32), jnp.int32)
    for i in range(k + 1):
        idx = jnp.argmin(d, axis=1).astype(jnp.int32)[:, None]
        d = jnp.where(cit == idx, jnp.inf, d)
        if i > 0:
            ni_acc = jnp.where(lane == i - 1, idx, ni_acc)
    return ni_acc[:, :k]


def _k1_body(vb_ref, va_ref, vt_ref, d0_ref, ni_ref, fm0_ref):
    vb = vb_ref[0]
    va = va_ref[0]
    vt = vt_ref[0]
    rows, cols = vb.shape[0], va.shape[0]
    d = _pdist(vb, vt)
    cit = lax.broadcasted_iota(jnp.int32, (rows, cols), 1)
    lane = lax.broadcasted_iota(jnp.int32, (rows, 32), 1)
    d0n = _norm_dirs(d0_ref[...])

    ni_acc = jnp.zeros((rows, 32), jnp.int32)
    acc0 = jnp.zeros((rows, 16), _F32)
    for i in range(_K + 1):
        m = jnp.min(d, axis=1, keepdims=True)
        idx = jnp.min(jnp.where(d == m, cit, cols), axis=1, keepdims=True)
        sel = cit == idx
        d = jnp.where(sel, jnp.inf, d)
        if i > 0:
            oh = sel.astype(_F32)
            nbr = _mm(oh, va)
            ndn = _norm_rows(nbr - vb)
            acc0 = jnp.maximum(acc0, _theta(ndn, d0n))
            ni_acc = jnp.where(lane == i - 1, idx, ni_acc)
    ni_ref[0] = ni_acc[:, :_K]
    fm0_ref[0] = acc0


def _k2_body(va_ref, vb_ref, fm0a_ref, fm0b_ref, ni_ref, w_ref, b_ref,
             dir_ref, fm1_ref):
    va = va_ref[0]
    vb = vb_ref[0]
    fa = fm0a_ref[0]
    fb = fm0b_ref[0]
    nib = ni_ref[0]
    w = w_ref[...]
    b = b_ref[...]
    sup_table = _mm(fa, w[:, 32:]) + b[:, 32:]
    center = _mm(fb, w[:, :32]) + b[:, :32]
    dn = _norm_dirs(dir_ref[...])
    table = jnp.concatenate([sup_table, va], axis=1)
    acc, _ = _conv_neighbor_max(nib, table, vb, dn, 32)
    fm1_ref[0] = jnp.maximum(center + acc, 0.0)


def _k4_body(vb_ref, vt_ref, fm1_ref, pni_ref, w2_ref, b2_ref, dir2_ref,
             w3_ref, b3_ref, dir3_ref, fm3_ref, ni2_ref):
    vb = vb_ref[0]
    vt = vt_ref[0]
    fm1 = fm1_ref[0]
    pni = pni_ref[0]

    fmp = _pool_max(pni, fm1)
    ni = _topk_ni(_pdist(vb, vt), _K)

    t2 = _mm(fmp, w2_ref[...]) + b2_ref[...]
    d2n = _norm_dirs(dir2_ref[...])
    tbl2 = jnp.concatenate([t2[:, 64:], vb], axis=1)
    acc, ndns = _conv_neighbor_max(ni, tbl2, vb, d2n, 64)
    fm2 = jnp.maximum(t2[:, :64] + acc, 0.0)

    mp2 = jnp.max(fm2, axis=0, keepdims=True)
    fm2c = jnp.concatenate([fm2, jnp.broadcast_to(mp2, fm2.shape)], axis=1)

    t3 = _mm(fm2c, w3_ref[...]) + b3_ref[...]
    d3n = _norm_dirs(dir3_ref[...])
    acc, _ = _conv_neighbor_max(ni, t3[:, 256:], vb, d3n, 256, ndns=ndns)
    fm3_ref[0] = jnp.maximum(t3[:, :256] + acc, 0.0)
    ni2_ref[0] = ni


def _k5_body(vb_ref, vt_ref, fm3_ref, pni_ref, w4_ref, b4_ref, dir4_ref,
             w5_ref, b5_ref, dir5_ref, fg_ref):
    vb = vb_ref[0]
    vt = vt_ref[0]
    fm3 = fm3_ref[0]
    pni = pni_ref[0]

    fmp = _pool_max(pni, fm3)
    ni = _topk_ni(_pdist(vb, vt), _K)

    t4 = _mm(fmp, w4_ref[...]) + b4_ref[...]
    d4n = _norm_dirs(dir4_ref[...])
    tbl4 = jnp.concatenate([t4[:, 256:], vb], axis=1)
    acc, ndns = _conv_neighbor_max(ni, tbl4, vb, d4n, 256)
    fm4 = jnp.maximum(t4[:, :256] + acc, 0.0)

    mp4 = jnp.max(fm4, axis=0, keepdims=True)
    fm4c = jnp.concatenate([fm4, jnp.broadcast_to(mp4, fm4.shape)], axis=1)

    t5 = _mm(fm4c, w5_ref[...]) + b5_ref[...]
    d5n = _norm_dirs(dir5_ref[...])
    acc, _ = _conv_neighbor_max(ni, t5[:, 1024:], vb, d5n, 1024, ndns=ndns)
    fm5 = jnp.maximum(t5[:, :1024] + acc, 0.0)
    fg_ref[0] = jnp.max(fm5, axis=0, keepdims=True)


def _k6_body(fg_ref, cW1_ref, cb1_ref, cg_ref, cbe_ref, cW2_ref, cb2_ref,
             dW1_ref, db1_ref, dg_ref, dbe_ref, dW2_ref, db2_ref,
             co_ref, do_ref):
    fg = fg_ref[...]
    inv = 1.0 / math.sqrt(1.0 + 1e-5)

    def _head(W1, b1, g, be, W2, b2):
        h = _mm(fg, W1) + b1
        h = g * h * inv + be
        h = jnp.maximum(h, 0.0)
        lg = _mm(h, W2) + b2
        m = jnp.max(lg, axis=1, keepdims=True)
        s = lg - m
        return s - jnp.log(jnp.sum(jnp.exp(s), axis=1, keepdims=True))

    co_ref[...] = _head(cW1_ref[...], cb1_ref[...], cg_ref[...],
                        cbe_ref[...], cW2_ref[...], cb2_ref[...])
    do_ref[...] = _head(dW1_ref[...], db1_ref[...], dg_ref[...],
                        dbe_ref[...], dW2_ref[...], db2_ref[...])


_PERM1 = np.array([
    1308,98,1494,1367,1392,726,410,1311,1631,1841,360,1261,1990,139,467,
    1964,1122,1547,739,892,198,610,1721,1669,1822,1265,1502,1965,858,292,
    210,965,1029,1185,1888,1968,688,1230,941,158,352,539,294,795,26,919,
    120,853,216,340,1356,1324,1164,236,13,482,414,1168,1726,1854,873,883,
    1909,1982,73,90,107,953,114,752,1388,1274,1556,702,88,226,868,1707,49,
    488,1761,1248,423,442,641,1767,1755,1012,1570,1598,0,1111,855,1142,
    1713,601,529,34,1522,1187,305,1087,202,948,751,443,806,206,1067,803,
    637,250,1224,51,1147,1772,533,457,661,1402,863,242,1534,1366,666,1756,
    1445,622,709,437,519,142,1847,1658,95,1700,1863,1381,1042,991,75,357,
    794,1549,495,1614,1451,525,1262,1030,1925,1904,404,1680,1942,200,385,
    1134,239,2003,39,619,1327,459,680,1475,432,694,1518,141,588,685,1660,
    122,715,1783,35,1139,274,797,1346,608,670,2001,362,409,1428,978,658,
    1543,1341,1343,1708,958,1843,1440,1406,378,1719,341,123,1306,116,1107,
    1967,21,1781,1896,1056,1026,551,1450,1926,1711,370,649,268,307,2034,
    2011,168,1500,1739,2000,1218,503,1325,748,1616,1193,1605,1437,1319,
    1595,1427,252,1481,1851,1116,1102,902,4,1053,273,1098,600,1453,386,
    1927,1734,1859,1974,1221,1683,763,1532,1724,365,829,732,1277,1831,
    1439,586,890,1836,96,1656,581,230,900,1943,1498,416,1,1794,1106,152,
    520,827,969,1206,245,1624,1741,452,1803,129,549,76,924,857,1931,884,
    623,1174,558,862,1826,315,448,361,754,1559,568,1586,254,1035,952,81,
    769,41,1144,2018,501,248,1268,382,575,1899,1104,2019,1213,1489,338,
    1045,973,280,1121,255,1099,1579,954,1555,1061,921,89,1090,1569,422,
    1635,400,93,1241,1373,407,1079,205,209,363,1988,839,636,871,647,1796,
    698,1048,615,218,1186,894,434,1393,767,1088,672,1084,47,692,293,66,
    1845,70,756,174,222,1457,2014,532,1520,1821,1645,1077,1488,1149,793,
    1097,1001,1671,1618,1505,1811,1156,387,1685,1674,426,1008,128,617,882,
    980,648,1524,1996,1938,1597,194,1834,1467,1949,1289,1743,312,1833,
    1615,1305,1027,1095,1177,598,1212,393,1897,1986,1485,917,285,1940,321,
    347,1566,950,1966,1062,611,728,1257,11,1426,1307,1676,435,1873,984,
    1696,1083,1215,741,1960,625,1419,845,1345,1535,308,309,1171,572,779,
    785,1571,824,557,1916,1359,578,156,771,440,1058,430,706,1805,9,1123,
    1023,1145,244,1663,1161,1878,1934,1880,1483,1544,997,1234,681,1094,
    727,1179,1376,1953,492,1499,995,718,736,333,1792,1390,1000,1868,1253,
    1205,957,1014,345,787,961,
], dtype=np.int32)
_PERM2 = np.array([
    135,164,319,83,387,107,91,503,52,58,2,379,450,238,156,501,59,467,73,
    15,388,177,449,375,394,498,284,225,53,129,243,136,415,196,63,10,484,
    239,359,455,185,444,244,497,158,181,198,422,474,138,113,393,67,29,389,
    94,396,162,456,62,163,499,260,468,464,159,229,311,179,271,248,174,191,
    273,426,241,92,224,365,117,295,383,391,126,446,505,508,251,110,459,98,
    309,81,451,441,373,352,250,66,476,349,438,285,431,482,55,478,343,249,
    294,9,85,28,469,194,124,259,448,80,386,18,480,235,176,45,31,408,418,
], dtype=np.int32)


def _full(shape):
    return pl.BlockSpec(shape, lambda b, *_: (0,) * len(shape))


def kernel(vertices, alpha, d0, w1, b1, dir1, w2, b2, dir2, w3, b3, dir3,
           w4, b4, dir4, w5, b5, dir5, cW1, cb1, cg, cbe, cW2, cb2,
           dW1, db1, dg, dbe, dW2, db2):
    del alpha
    bs, V, _ = vertices.shape
    Rb = 256
    nb = V // Rb
    f32 = jnp.float32
    vT = jnp.swapaxes(vertices, 1, 2)
    b1r, b2r, b3r, b4r, b5r = (x.reshape(1, -1) for x in (b1, b2, b3, b4, b5))

    ni1, fm0 = pl.pallas_call(
        _k1_body,
        grid=(bs, nb),
        in_specs=[
            pl.BlockSpec((1, Rb, 3), lambda b, r: (b, r, 0)),
            pl.BlockSpec((1, V, 3), lambda b, r: (b, 0, 0)),
            pl.BlockSpec((1, 3, V), lambda b, r: (b, 0, 0)),
            pl.BlockSpec((3, 16), lambda b, r: (0, 0)),
        ],
        out_specs=[
            pl.BlockSpec((1, Rb, _K), lambda b, r: (b, r, 0)),
            pl.BlockSpec((1, Rb, 16), lambda b, r: (b, r, 0)),
        ],
        out_shape=[
            jax.ShapeDtypeStruct((bs, V, _K), jnp.int32),
            jax.ShapeDtypeStruct((bs, V, 16), f32),
        ],
    )(vertices, vertices, vT, d0)

    fm1 = pl.pallas_call(
        _k2_body,
        grid=(bs, nb),
        in_specs=[
            pl.BlockSpec((1, V, 3), lambda b, r: (b, 0, 0)),
            pl.BlockSpec((1, Rb, 3), lambda b, r: (b, r, 0)),
            pl.BlockSpec((1, V, 16), lambda b, r: (b, 0, 0)),
            pl.BlockSpec((1, Rb, 16), lambda b, r: (b, r, 0)),
            pl.BlockSpec((1, Rb, _K), lambda b, r: (b, r, 0)),
            pl.BlockSpec((16, 64), lambda b, r: (0, 0)),
            pl.BlockSpec((1, 64), lambda b, r: (0, 0)),
            pl.BlockSpec((3, 32), lambda b, r: (0, 0)),
        ],
        out_specs=pl.BlockSpec((1, Rb, 32), lambda b, r: (b, r, 0)),
        out_shape=jax.ShapeDtypeStruct((bs, V, 32), f32),
    )(vertices, vertices, fm0, fm0, ni1, w1, b1r, dir1)

    V2 = 512
    vert2 = vertices[:, _PERM1, :]
    v2T = jnp.swapaxes(vert2, 1, 2)
    pni1 = ni1[:, _PERM1, :4]

    fm3, ni2 = pl.pallas_call(
        _k4_body,
        grid=(bs,),
        in_specs=[
            pl.BlockSpec((1, V2, 3), lambda b: (b, 0, 0)),
            pl.BlockSpec((1, 3, V2), lambda b: (b, 0, 0)),
            pl.BlockSpec((1, V, 32), lambda b: (b, 0, 0)),
            pl.BlockSpec((1, V2, 4), lambda b: (b, 0, 0)),
            _full((32, 128)), _full((1, 128)), _full((3, 64)),
            _full((128, 512)), _full((1, 512)), _full((3, 256)),
        ],
        out_specs=[
            pl.BlockSpec((1, V2, 256), lambda b: (b, 0, 0)),
            pl.BlockSpec((1, V2, _K), lambda b: (b, 0, 0)),
        ],
        out_shape=[
            jax.ShapeDtypeStruct((bs, V2, 256), f32),
            jax.ShapeDtypeStruct((bs, V2, _K), jnp.int32),
        ],
    )(vert2, v2T, fm1, pni1, w2, b2r, dir2, w3, b3r, dir3)

    V3 = 128
    vert3 = vert2[:, _PERM2, :]
    v3T = jnp.swapaxes(vert3, 1, 2)
    pni2 = ni2[:, _PERM2, :4]

    fg = pl.pallas_call(
        _k5_body,
        grid=(bs,),
        in_specs=[
            pl.BlockSpec((1, V3, 3), lambda b: (b, 0, 0)),
            pl.BlockSpec((1, 3, V3), lambda b: (b, 0, 0)),
            pl.BlockSpec((1, V2, 256), lambda b: (b, 0, 0)),
            pl.BlockSpec((1, V3, 4), lambda b: (b, 0, 0)),
            _full((256, 512)), _full((1, 512)), _full((3, 256)),
            _full((512, 2048)), _full((1, 2048)), _full((3, 1024)),
        ],
        out_specs=pl.BlockSpec((1, 1, 1024), lambda b: (b, 0, 0)),
        out_shape=jax.ShapeDtypeStruct((bs, 1, 1024), f32),
    )(vert3, v3T, fm3, pni2, w4, b4r, dir4, w5, b5r, dir5)

    fgr = fg.reshape(bs, 1024)
    co, do = pl.pallas_call(
        _k6_body,
        in_specs=[
            pl.BlockSpec((bs, 1024), lambda: (0, 0)),
            pl.BlockSpec((1024, 256), lambda: (0, 0)),
            pl.BlockSpec((1, 256), lambda: (0, 0)),
            pl.BlockSpec((1, 256), lambda: (0, 0)),
            pl.BlockSpec((1, 256), lambda: (0, 0)),
            pl.BlockSpec((256, 3), lambda: (0, 0)),
            pl.BlockSpec((1, 3), lambda: (0, 0)),
            pl.BlockSpec((1024, 256), lambda: (0, 0)),
            pl.BlockSpec((1, 256), lambda: (0, 0)),
            pl.BlockSpec((1, 256), lambda: (0, 0)),
            pl.BlockSpec((1, 256), lambda: (0, 0)),
            pl.BlockSpec((256, 2), lambda: (0, 0)),
            pl.BlockSpec((1, 2), lambda: (0, 0)),
        ],
        out_specs=[
            pl.BlockSpec((bs, 3), lambda: (0, 0)),
            pl.BlockSpec((bs, 2), lambda: (0, 0)),
        ],
        out_shape=[
            jax.ShapeDtypeStruct((bs, 3), f32),
            jax.ShapeDtypeStruct((bs, 2), f32),
        ],
    )(fgr, cW1, cb1.reshape(1, -1), cg.reshape(1, -1), cbe.reshape(1, -1),
      cW2, cb2.reshape(1, -1), dW1, db1.reshape(1, -1), dg.reshape(1, -1),
      dbe.reshape(1, -1), dW2, db2.reshape(1, -1))
    return co, do
